# Initial kernel scaffold; baseline (speedup 1.0000x reference)
import jax
import jax.numpy as jnp
from jax.experimental import pallas as pl


def kernel(xs, A_to_B_edge_idx, B_to_A_edge_idx, W_max, b_max, W_conv, b_conv, bn_fw_gamma, bn_fw_beta, bn_bw_gamma, bn_bw_beta, prelu_a):
    raise NotImplementedError("write your pallas kernel here")


# speedup vs baseline: 2.9510x; 2.9510x over previous
import functools

import jax
import jax.numpy as jnp
from jax.experimental import pallas as pl
from jax.experimental.pallas import tpu as pltpu

_N = 10000


def _bn_prelu_kernel(y_ref, gam_ref, bet_ref, a_ref, o_ref):
    d = pl.program_id(0)
    g_id = pl.program_id(1)
    blk = y_ref[0, 0]
    mu = jnp.mean(blk)
    var = jnp.mean(blk * blk) - mu * mu
    g = gam_ref[d, g_id]
    b = bet_ref[d, g_id]
    yn = g * (blk - mu) * jax.lax.rsqrt(var + 1e-5) + b
    a = a_ref[0]
    o_ref[0, 0] = jnp.where(yn >= 0, yn, a * yn)


def _bn_prelu(y, gamma, beta, prelu_a):
    D, E, C = y.shape
    bs = E // C
    y4 = y.reshape(D, C, bs, C)
    out = pl.pallas_call(
        _bn_prelu_kernel,
        grid=(D, C),
        in_specs=[
            pl.BlockSpec((1, 1, bs, C), lambda d, g: (d, g, 0, 0)),
            pl.BlockSpec(memory_space=pltpu.SMEM),
            pl.BlockSpec(memory_space=pltpu.SMEM),
            pl.BlockSpec(memory_space=pltpu.SMEM),
        ],
        out_specs=pl.BlockSpec((1, 1, bs, C), lambda d, g: (d, g, 0, 0)),
        out_shape=jax.ShapeDtypeStruct((D, C, bs, C), jnp.float32),
    )(y4, gamma, beta, prelu_a.reshape(1))
    return out.reshape(D, E, C)


def kernel(xs, A_to_B_edge_idx, B_to_A_edge_idx, W_max, b_max, W_conv, b_conv,
           bn_fw_gamma, bn_fw_beta, bn_bw_gamma, bn_bw_beta, prelu_a):
    E = xs.shape[2]
    x0 = xs[0, 0]
    x1 = xs[1, 0]
    a0 = A_to_B_edge_idx[0, 0]
    a1 = A_to_B_edge_idx[0, 1]
    b0 = B_to_A_edge_idx[0, 0]
    b1 = B_to_A_edge_idx[0, 1]

    ones = jnp.ones((E,), jnp.float32)
    s0 = jax.ops.segment_sum(x0, a0, num_segments=_N)
    c0 = jax.ops.segment_sum(ones, a0, num_segments=_N)
    s1 = jax.ops.segment_sum(x1, b0, num_segments=_N)
    c1 = jax.ops.segment_sum(ones, b0, num_segments=_N)
    m0 = s0 / jnp.clip(c0, 1.0)[:, None]
    m1 = s1 / jnp.clip(c1, 1.0)[:, None]

    F = x0.shape[1]
    Wm1, Wm2 = W_max[:, :F], W_max[:, F:]
    t1m = m1 @ Wm2.T + b_max
    t0m = m0 @ Wm2.T + b_max

    u0 = x0 @ Wm1.T
    u1 = x1 @ Wm1.T
    z_fw = u0 + t1m[a1]
    z_bw = u1 + t0m[b1]

    zmax_fw = jax.ops.segment_max(z_fw, a0, num_segments=_N)
    zmax_bw = jax.ops.segment_max(z_bw, b0, num_segments=_N)
    zmax_fw = jnp.where(c0[:, None] > 0, zmax_fw, 0.0)
    zmax_bw = jnp.where(c1[:, None] > 0, zmax_bw, 0.0)

    Wc1, Wc2, Wc3 = W_conv[:, :F], W_conv[:, F:2 * F], W_conv[:, 2 * F:]
    v0 = x0 @ Wc1.T + b_conv
    v1 = x1 @ Wc1.T + b_conv
    t1c = m1 @ Wc2.T
    t0c = m0 @ Wc2.T
    sf = zmax_fw @ Wc3.T
    sb = zmax_bw @ Wc3.T

    out_fw = v0 + t1c[a1] + sf[a0]
    out_bw = v1 + t0c[b1] + sb[b0]

    y = jnp.stack([out_fw, out_bw], axis=0)
    y = _bn_prelu(y, jnp.stack([bn_fw_gamma, bn_bw_gamma]),
                  jnp.stack([bn_fw_beta, bn_bw_beta]), prelu_a)
    return y[:, None]


# speedup vs baseline: 4.3564x; 1.4762x over previous
import functools

import jax
import jax.numpy as jnp
from jax import lax
from jax.experimental import pallas as pl
from jax.experimental.pallas import tpu as pltpu
from jax.experimental.pallas import tpu_sc as plsc

_N = 10000
_NC = 2
_NS = 16


def _scmean_body(x2, i2, ones_hbm, zf_hbm, zn_hbm,
                 sums_out, cnts_out,
                 xbuf, ibuf, ones_v, acc, cnt):
    E = x2.shape[1]
    epw = E // _NS
    W = 1000
    nwin = epw // W
    c = lax.axis_index("c")
    s = lax.axis_index("s")

    pltpu.sync_copy(ones_hbm, ones_v)
    @pl.when(s == 0)
    def _init():
        pltpu.sync_copy(zf_hbm, acc)
        pltpu.sync_copy(zn_hbm, cnt)
    plsc.subcore_barrier()

    base = s * epw
    for w in range(nwin):
        off = base + w * W
        pltpu.sync_copy(i2.at[c, pl.ds(off, W)], ibuf)
        pltpu.sync_copy(x2.at[c, pl.ds(off, W), :], xbuf)
        pltpu.sync_copy(xbuf, acc.at[ibuf], add=True)
        pltpu.sync_copy(ones_v, cnt.at[ibuf], add=True)
    plsc.subcore_barrier()

    @pl.when(s == 0)
    def _flush():
        pltpu.sync_copy(acc, sums_out.at[c])
        pltpu.sync_copy(cnt, cnts_out.at[c])


def _scmean(x2, a0, b0):
    _, E, F = x2.shape
    W = 1000
    ones_hbm = jnp.ones((W,), jnp.float32)
    zf_hbm = jnp.zeros((_N, F), jnp.float32)
    zn_hbm = jnp.zeros((_N,), jnp.float32)
    i2 = jnp.stack([a0, b0])
    k = pl.kernel(
        _scmean_body,
        out_type=[jax.ShapeDtypeStruct((2, _N, F), jnp.float32),
                  jax.ShapeDtypeStruct((2, _N), jnp.float32)],
        mesh=plsc.VectorSubcoreMesh(core_axis_name="c", subcore_axis_name="s"),
        compiler_params=pltpu.CompilerParams(use_tc_tiling_on_sc=False),
        scratch_types=[
            pltpu.VMEM((W, F), jnp.float32),
            pltpu.VMEM((W,), jnp.int32),
            pltpu.VMEM((W,), jnp.float32),
            pltpu.VMEM_SHARED((_N, F), jnp.float32),
            pltpu.VMEM_SHARED((_N,), jnp.float32),
        ],
    )
    sums, cnts = k(x2, i2, ones_hbm, zf_hbm, zn_hbm)
    return sums[0], sums[1], cnts[0], cnts[1]


def _scmax_body(u_fbt, tm_fb, ig2, im2, ninf_hbm, zpart_out,
                ubuf, gbuf, ib1, ib0, acc):
    E = ig2.shape[1]
    Q = E // 4
    W = 640
    nwin = Q // W
    ng = W // 16
    c = lax.axis_index("c")
    s = lax.axis_index("s")
    fg = s % 8
    eg = c * 2 + s // 8
    iota = lax.iota(jnp.int32, 16)
    fsp = [jnp.full((16,), f, jnp.int32) for f in range(8)]

    for d in range(2):
        pltpu.sync_copy(ninf_hbm, acc)
        qbase = eg * Q

        def _win(w):
            off = qbase + w * W
            pltpu.sync_copy(ig2.at[d, pl.ds(off, W)], ib1)
            pltpu.sync_copy(im2.at[d, pl.ds(off, W)], ib0)
            pltpu.sync_copy(u_fbt.at[d, fg, :, pl.ds(off, W)], ubuf)
            pltpu.sync_copy(tm_fb.at[d, fg].at[ib1], gbuf)

            def _grp(j):
                b16 = j * 16
                a0v = ib0[pl.ds(b16, 16)]
                rows = iota + b16
                zs = []
                for f in range(8):
                    zt = plsc.load_gather(gbuf, [rows, fsp[f]])
                    uv = ubuf[f, pl.ds(b16, 16)]
                    zs.append(zt + uv)

                def _cond(pending):
                    return jnp.any(pending)

                def _body(pending):
                    fail = jnp.zeros((16,), jnp.bool_)
                    for f in range(8):
                        cur = plsc.load_gather(acc, [a0v, fsp[f]])
                        mx = jnp.maximum(cur, zs[f])
                        plsc.store_scatter(acc, [a0v, fsp[f]], mx, mask=pending)
                        chk = plsc.load_gather(acc, [a0v, fsp[f]])
                        fail = fail | (chk < mx)
                    return pending & fail

                lax.while_loop(_cond, _body, jnp.ones((16,), jnp.bool_))

            lax.fori_loop(0, ng, lambda j, _: (_grp(j), 0)[1], 0)

        lax.fori_loop(0, nwin, lambda w, _: (_win(w), 0)[1], 0)
        pltpu.sync_copy(acc, zpart_out.at[d, eg, fg])


def _scmax(u_fbt, tm_fb, ig2, im2):
    E = ig2.shape[1]
    W = 640
    ninf_hbm = jnp.full((_N, 8), -jnp.inf, jnp.float32)
    k = pl.kernel(
        _scmax_body,
        out_type=jax.ShapeDtypeStruct((2, 4, 8, _N, 8), jnp.float32),
        mesh=plsc.VectorSubcoreMesh(core_axis_name="c", subcore_axis_name="s"),
        compiler_params=pltpu.CompilerParams(use_tc_tiling_on_sc=False,
                                             needs_layout_passes=False),
        scratch_types=[
            pltpu.VMEM((8, W), jnp.float32),
            pltpu.VMEM((W, 8), jnp.float32),
            pltpu.VMEM((W,), jnp.int32),
            pltpu.VMEM((W,), jnp.int32),
            pltpu.VMEM((_N, 8), jnp.float32),
        ],
    )
    return k(u_fbt, tm_fb, ig2, im2, ninf_hbm)


def _scg_body(tc2, sc2, ig2, im2, g_out, s_out, gb, sb, ibg, ibm):
    E = ig2.shape[1]
    epw = E // _NS
    W = 400
    nwin = epw // W
    c = lax.axis_index("c")
    s = lax.axis_index("s")
    base = s * epw

    def _win(w):
        off = base + w * W
        pltpu.sync_copy(ig2.at[c, pl.ds(off, W)], ibg)
        pltpu.sync_copy(im2.at[c, pl.ds(off, W)], ibm)
        pltpu.sync_copy(tc2.at[c].at[ibg], gb)
        pltpu.sync_copy(sc2.at[c].at[ibm], sb)
        pltpu.sync_copy(gb, g_out.at[c, pl.ds(off, W), :])
        pltpu.sync_copy(sb, s_out.at[c, pl.ds(off, W), :])

    lax.fori_loop(0, nwin, lambda w, _: (_win(w), 0)[1], 0)


def _scgather2(tc2, sc2, ig2, im2):
    E = ig2.shape[1]
    C = tc2.shape[2]
    W = 400
    k = pl.kernel(
        _scg_body,
        out_type=[jax.ShapeDtypeStruct((2, E, C), jnp.float32),
                  jax.ShapeDtypeStruct((2, E, C), jnp.float32)],
        mesh=plsc.VectorSubcoreMesh(core_axis_name="c", subcore_axis_name="s"),
        compiler_params=pltpu.CompilerParams(use_tc_tiling_on_sc=False,
                                             needs_layout_passes=False),
        scratch_types=[
            pltpu.VMEM((W, C), jnp.float32),
            pltpu.VMEM((W, C), jnp.float32),
            pltpu.VMEM((W,), jnp.int32),
            pltpu.VMEM((W,), jnp.int32),
        ],
    )
    return k(tc2, sc2, ig2, im2)


def _conv_bn_prelu_kernel(x_ref, g_ref, s_ref, w_ref, bc_ref, gam_ref,
                          bet_ref, a_ref, o_ref):
    d = pl.program_id(0)
    g_id = pl.program_id(1)
    xb = x_ref[0, 0]
    y = lax.dot_general(xb, w_ref[...], (((1,), (1,)), ((), ())),
                        preferred_element_type=jnp.float32)
    y = y + bc_ref[...]
    y = y + g_ref[0, 0] + s_ref[0, 0]
    mu = jnp.mean(y)
    var = jnp.mean(y * y) - mu * mu
    g = gam_ref[d, g_id]
    b = bet_ref[d, g_id]
    yn = g * (y - mu) * jax.lax.rsqrt(var + 1e-5) + b
    a = a_ref[0]
    o_ref[0, 0] = jnp.where(yn >= 0, yn, a * yn)


def _conv_bn_prelu(x2, G, S, Wc1, b_conv, gamma, beta, prelu_a):
    D, E, F = x2.shape
    C = G.shape[2]
    bs = E // C
    x4 = x2.reshape(D, C, bs, F)
    G4 = G.reshape(D, C, bs, C)
    S4 = S.reshape(D, C, bs, C)
    out = pl.pallas_call(
        _conv_bn_prelu_kernel,
        grid=(D, C),
        in_specs=[
            pl.BlockSpec((1, 1, bs, F), lambda d, g: (d, g, 0, 0)),
            pl.BlockSpec((1, 1, bs, C), lambda d, g: (d, g, 0, 0)),
            pl.BlockSpec((1, 1, bs, C), lambda d, g: (d, g, 0, 0)),
            pl.BlockSpec((C, F), lambda d, g: (0, 0)),
            pl.BlockSpec((1, C), lambda d, g: (0, 0)),
            pl.BlockSpec(memory_space=pltpu.SMEM),
            pl.BlockSpec(memory_space=pltpu.SMEM),
            pl.BlockSpec(memory_space=pltpu.SMEM),
        ],
        out_specs=pl.BlockSpec((1, 1, bs, C), lambda d, g: (d, g, 0, 0)),
        out_shape=jax.ShapeDtypeStruct((D, C, bs, C), jnp.float32),
    )(x4, G4, S4, Wc1, b_conv.reshape(1, C), gamma, beta, prelu_a.reshape(1))
    return out.reshape(D, E, C)


def kernel(xs, A_to_B_edge_idx, B_to_A_edge_idx, W_max, b_max, W_conv, b_conv,
           bn_fw_gamma, bn_fw_beta, bn_bw_gamma, bn_bw_beta, prelu_a):
    E = xs.shape[2]
    x0 = xs[0, 0]
    x1 = xs[1, 0]
    a0 = A_to_B_edge_idx[0, 0]
    a1 = A_to_B_edge_idx[0, 1]
    b0 = B_to_A_edge_idx[0, 0]
    b1 = B_to_A_edge_idx[0, 1]

    s0, s1, c0, c1 = _scmean(xs.reshape(2, E, -1), a0, b0)
    m0 = s0 / jnp.clip(c0, 1.0)[:, None]
    m1 = s1 / jnp.clip(c1, 1.0)[:, None]

    F = x0.shape[1]
    Wm1, Wm2 = W_max[:, :F], W_max[:, F:]
    t1m = m1 @ Wm2.T + b_max
    t0m = m0 @ Wm2.T + b_max

    u = jnp.einsum('def,gf->dge', xs.reshape(2, E, -1), Wm1)
    u_fbt = u.reshape(2, 8, 8, E)
    tm = jnp.stack([t1m, t0m])
    tm_fb = tm.reshape(2, _N, 8, 8).transpose(0, 2, 1, 3)
    ig2 = jnp.stack([a1, b1])
    im2 = jnp.stack([a0, b0])
    zpart = _scmax(u_fbt, tm_fb, ig2, im2)
    zmax = zpart.max(axis=1).transpose(0, 2, 1, 3).reshape(2, _N, 64)
    cnt2 = jnp.stack([c0, c1])
    zmax = jnp.where(cnt2[:, :, None] > 0, zmax, 0.0)
    zmax_fw, zmax_bw = zmax[0], zmax[1]

    Wc1, Wc2, Wc3 = W_conv[:, :F], W_conv[:, F:2 * F], W_conv[:, 2 * F:]
    t1c = m1 @ Wc2.T
    t0c = m0 @ Wc2.T
    sf = zmax_fw @ Wc3.T
    sb = zmax_bw @ Wc3.T

    tc2 = jnp.stack([t1c, t0c])
    sc2 = jnp.stack([sf, sb])
    G, S = _scgather2(tc2, sc2, ig2, im2)

    y = _conv_bn_prelu(xs.reshape(2, E, -1), G, S, Wc1, b_conv,
                       jnp.stack([bn_fw_gamma, bn_bw_gamma]),
                       jnp.stack([bn_fw_beta, bn_bw_beta]), prelu_a)
    return y[:, None]


# speedup vs baseline: 4.4437x; 1.0200x over previous
import functools

import jax
import jax.numpy as jnp
from jax import lax
from jax.experimental import pallas as pl
from jax.experimental.pallas import tpu as pltpu
from jax.experimental.pallas import tpu_sc as plsc

_N = 10000
_NC = 2
_NS = 16


def _scmean_body(x2, i2, ones_hbm, zf_hbm, zn_hbm,
                 sums_out, cnts_out,
                 xbuf, ibuf, ones_v, acc, cnt):
    E = x2.shape[1]
    epw = E // _NS
    W = 1000
    nwin = epw // W
    c = lax.axis_index("c")
    s = lax.axis_index("s")

    pltpu.sync_copy(ones_hbm, ones_v)
    @pl.when(s == 0)
    def _init():
        pltpu.sync_copy(zf_hbm, acc)
        pltpu.sync_copy(zn_hbm, cnt)
    plsc.subcore_barrier()

    base = s * epw
    for w in range(nwin):
        off = base + w * W
        pltpu.sync_copy(i2.at[c, pl.ds(off, W)], ibuf)
        pltpu.sync_copy(x2.at[c, pl.ds(off, W), :], xbuf)
        pltpu.sync_copy(xbuf, acc.at[ibuf], add=True)
        pltpu.sync_copy(ones_v, cnt.at[ibuf], add=True)
    plsc.subcore_barrier()

    @pl.when(s == 0)
    def _flush():
        pltpu.sync_copy(acc, sums_out.at[c])
        pltpu.sync_copy(cnt, cnts_out.at[c])


def _scmean(x2, a0, b0):
    _, E, F = x2.shape
    W = 1000
    ones_hbm = jnp.ones((W,), jnp.float32)
    zf_hbm = jnp.zeros((_N, F), jnp.float32)
    zn_hbm = jnp.zeros((_N,), jnp.float32)
    i2 = jnp.stack([a0, b0])
    k = pl.kernel(
        _scmean_body,
        out_type=[jax.ShapeDtypeStruct((2, _N, F), jnp.float32),
                  jax.ShapeDtypeStruct((2, _N), jnp.float32)],
        mesh=plsc.VectorSubcoreMesh(core_axis_name="c", subcore_axis_name="s"),
        compiler_params=pltpu.CompilerParams(use_tc_tiling_on_sc=False),
        scratch_types=[
            pltpu.VMEM((W, F), jnp.float32),
            pltpu.VMEM((W,), jnp.int32),
            pltpu.VMEM((W,), jnp.float32),
            pltpu.VMEM_SHARED((_N, F), jnp.float32),
            pltpu.VMEM_SHARED((_N,), jnp.float32),
        ],
    )
    sums, cnts = k(x2, i2, ones_hbm, zf_hbm, zn_hbm)
    return sums[0], sums[1], cnts[0], cnts[1]


def _scmax_body(u_fbt, tm_fb, ig2, im2, ninf_hbm, zpart_out,
                ubuf, gbuf, ib1, ib0, acc):
    E = ig2.shape[1]
    Q = E // 4
    W = 640
    nwin = Q // W
    ng = W // 16
    c = lax.axis_index("c")
    s = lax.axis_index("s")
    fg = s % 8
    eg = c * 2 + s // 8
    iota = lax.iota(jnp.int32, 16)
    im1 = jnp.maximum(iota - 1, 0)
    fsp = [jnp.full((16,), f, jnp.int32) for f in range(8)]

    for d in range(2):
        pltpu.sync_copy(ninf_hbm, acc)
        qbase = eg * Q

        def _win(w):
            off = qbase + w * W
            pltpu.sync_copy(ig2.at[d, pl.ds(off, W)], ib1)
            pltpu.sync_copy(im2.at[d, pl.ds(off, W)], ib0)
            pltpu.sync_copy(u_fbt.at[d, fg, :, pl.ds(off, W)], ubuf)
            pltpu.sync_copy(tm_fb.at[d, fg].at[ib1], gbuf)

            def _grp(j):
                b16 = j * 16
                a0v = ib0[pl.ds(b16, 16)]
                rows = iota + b16
                zs = []
                for f in range(8):
                    zt = plsc.load_gather(gbuf, [rows, fsp[f]])
                    uv = ubuf[f, pl.ds(b16, 16)]
                    zs.append(zt + uv)

                sk, _ = plsc.sort_key_val(a0v, a0v)
                skm1 = lax.gather(
                    sk, im1[:, None],
                    lax.GatherDimensionNumbers(offset_dims=(),
                                               collapsed_slice_dims=(0,),
                                               start_index_map=(0,)),
                    (1,), mode=lax.GatherScatterMode.PROMISE_IN_BOUNDS)
                dup = jnp.any((sk == skm1) & (iota > 0))

                @pl.when(jnp.logical_not(dup))
                def _fast():
                    for f in range(8):
                        cur = plsc.load_gather(acc, [a0v, fsp[f]])
                        plsc.store_scatter(acc, [a0v, fsp[f]],
                                           jnp.maximum(cur, zs[f]))

                @pl.when(dup)
                def _slow():
                    def _cond(pending):
                        return jnp.any(pending)

                    def _body(pending):
                        fail = jnp.zeros((16,), jnp.bool_)
                        for f in range(8):
                            cur = plsc.load_gather(acc, [a0v, fsp[f]])
                            mx = jnp.maximum(cur, zs[f])
                            plsc.store_scatter(acc, [a0v, fsp[f]], mx,
                                               mask=pending)
                            chk = plsc.load_gather(acc, [a0v, fsp[f]])
                            fail = fail | (chk < mx)
                        return pending & fail

                    lax.while_loop(_cond, _body, jnp.ones((16,), jnp.bool_))

            lax.fori_loop(0, ng, lambda j, _: (_grp(j), 0)[1], 0)

        lax.fori_loop(0, nwin, lambda w, _: (_win(w), 0)[1], 0)
        pltpu.sync_copy(acc, zpart_out.at[d, eg, fg])


def _scmax(u_fbt, tm_fb, ig2, im2):
    E = ig2.shape[1]
    W = 640
    ninf_hbm = jnp.full((_N, 8), -jnp.inf, jnp.float32)
    k = pl.kernel(
        _scmax_body,
        out_type=jax.ShapeDtypeStruct((2, 4, 8, _N, 8), jnp.float32),
        mesh=plsc.VectorSubcoreMesh(core_axis_name="c", subcore_axis_name="s"),
        compiler_params=pltpu.CompilerParams(use_tc_tiling_on_sc=False,
                                             needs_layout_passes=False),
        scratch_types=[
            pltpu.VMEM((8, W), jnp.float32),
            pltpu.VMEM((W, 8), jnp.float32),
            pltpu.VMEM((W,), jnp.int32),
            pltpu.VMEM((W,), jnp.int32),
            pltpu.VMEM((_N, 8), jnp.float32),
        ],
    )
    return k(u_fbt, tm_fb, ig2, im2, ninf_hbm)


def _scg_body(tc2, sc2, ig2, im2, g_out, s_out, gb, sb, ibg, ibm):
    E = ig2.shape[1]
    epw = E // _NS
    W = 400
    nwin = epw // W
    c = lax.axis_index("c")
    s = lax.axis_index("s")
    base = s * epw

    def _win(w):
        off = base + w * W
        pltpu.sync_copy(ig2.at[c, pl.ds(off, W)], ibg)
        pltpu.sync_copy(im2.at[c, pl.ds(off, W)], ibm)
        pltpu.sync_copy(tc2.at[c].at[ibg], gb)
        pltpu.sync_copy(sc2.at[c].at[ibm], sb)
        pltpu.sync_copy(gb, g_out.at[c, pl.ds(off, W), :])
        pltpu.sync_copy(sb, s_out.at[c, pl.ds(off, W), :])

    lax.fori_loop(0, nwin, lambda w, _: (_win(w), 0)[1], 0)


def _scgather2(tc2, sc2, ig2, im2):
    E = ig2.shape[1]
    C = tc2.shape[2]
    W = 400
    k = pl.kernel(
        _scg_body,
        out_type=[jax.ShapeDtypeStruct((2, E, C), jnp.float32),
                  jax.ShapeDtypeStruct((2, E, C), jnp.float32)],
        mesh=plsc.VectorSubcoreMesh(core_axis_name="c", subcore_axis_name="s"),
        compiler_params=pltpu.CompilerParams(use_tc_tiling_on_sc=False,
                                             needs_layout_passes=False),
        scratch_types=[
            pltpu.VMEM((W, C), jnp.float32),
            pltpu.VMEM((W, C), jnp.float32),
            pltpu.VMEM((W,), jnp.int32),
            pltpu.VMEM((W,), jnp.int32),
        ],
    )
    return k(tc2, sc2, ig2, im2)


def _conv_bn_prelu_kernel(x_ref, g_ref, s_ref, w_ref, bc_ref, gam_ref,
                          bet_ref, a_ref, o_ref):
    d = pl.program_id(0)
    g_id = pl.program_id(1)
    xb = x_ref[0, 0]
    y = lax.dot_general(xb, w_ref[...], (((1,), (1,)), ((), ())),
                        preferred_element_type=jnp.float32)
    y = y + bc_ref[...]
    y = y + g_ref[0, 0] + s_ref[0, 0]
    mu = jnp.mean(y)
    var = jnp.mean(y * y) - mu * mu
    g = gam_ref[d, g_id]
    b = bet_ref[d, g_id]
    yn = g * (y - mu) * jax.lax.rsqrt(var + 1e-5) + b
    a = a_ref[0]
    o_ref[0, 0] = jnp.where(yn >= 0, yn, a * yn)


def _conv_bn_prelu(x2, G, S, Wc1, b_conv, gamma, beta, prelu_a):
    D, E, F = x2.shape
    C = G.shape[2]
    bs = E // C
    x4 = x2.reshape(D, C, bs, F)
    G4 = G.reshape(D, C, bs, C)
    S4 = S.reshape(D, C, bs, C)
    out = pl.pallas_call(
        _conv_bn_prelu_kernel,
        grid=(D, C),
        in_specs=[
            pl.BlockSpec((1, 1, bs, F), lambda d, g: (d, g, 0, 0)),
            pl.BlockSpec((1, 1, bs, C), lambda d, g: (d, g, 0, 0)),
            pl.BlockSpec((1, 1, bs, C), lambda d, g: (d, g, 0, 0)),
            pl.BlockSpec((C, F), lambda d, g: (0, 0)),
            pl.BlockSpec((1, C), lambda d, g: (0, 0)),
            pl.BlockSpec(memory_space=pltpu.SMEM),
            pl.BlockSpec(memory_space=pltpu.SMEM),
            pl.BlockSpec(memory_space=pltpu.SMEM),
        ],
        out_specs=pl.BlockSpec((1, 1, bs, C), lambda d, g: (d, g, 0, 0)),
        out_shape=jax.ShapeDtypeStruct((D, C, bs, C), jnp.float32),
    )(x4, G4, S4, Wc1, b_conv.reshape(1, C), gamma, beta, prelu_a.reshape(1))
    return out.reshape(D, E, C)


def kernel(xs, A_to_B_edge_idx, B_to_A_edge_idx, W_max, b_max, W_conv, b_conv,
           bn_fw_gamma, bn_fw_beta, bn_bw_gamma, bn_bw_beta, prelu_a):
    E = xs.shape[2]
    x0 = xs[0, 0]
    x1 = xs[1, 0]
    a0 = A_to_B_edge_idx[0, 0]
    a1 = A_to_B_edge_idx[0, 1]
    b0 = B_to_A_edge_idx[0, 0]
    b1 = B_to_A_edge_idx[0, 1]

    s0, s1, c0, c1 = _scmean(xs.reshape(2, E, -1), a0, b0)
    m0 = s0 / jnp.clip(c0, 1.0)[:, None]
    m1 = s1 / jnp.clip(c1, 1.0)[:, None]

    F = x0.shape[1]
    Wm1, Wm2 = W_max[:, :F], W_max[:, F:]
    t1m = m1 @ Wm2.T + b_max
    t0m = m0 @ Wm2.T + b_max

    u = jnp.einsum('def,gf->dge', xs.reshape(2, E, -1), Wm1)
    u_fbt = u.reshape(2, 8, 8, E)
    tm = jnp.stack([t1m, t0m])
    tm_fb = tm.reshape(2, _N, 8, 8).transpose(0, 2, 1, 3)
    ig2 = jnp.stack([a1, b1])
    im2 = jnp.stack([a0, b0])
    zpart = _scmax(u_fbt, tm_fb, ig2, im2)
    zmax = zpart.max(axis=1).transpose(0, 2, 1, 3).reshape(2, _N, 64)
    cnt2 = jnp.stack([c0, c1])
    zmax = jnp.where(cnt2[:, :, None] > 0, zmax, 0.0)
    zmax_fw, zmax_bw = zmax[0], zmax[1]

    Wc1, Wc2, Wc3 = W_conv[:, :F], W_conv[:, F:2 * F], W_conv[:, 2 * F:]
    t1c = m1 @ Wc2.T
    t0c = m0 @ Wc2.T
    sf = zmax_fw @ Wc3.T
    sb = zmax_bw @ Wc3.T

    tc2 = jnp.stack([t1c, t0c])
    sc2 = jnp.stack([sf, sb])
    G, S = _scgather2(tc2, sc2, ig2, im2)

    y = _conv_bn_prelu(xs.reshape(2, E, -1), G, S, Wc1, b_conv,
                       jnp.stack([bn_fw_gamma, bn_bw_gamma]),
                       jnp.stack([bn_fw_beta, bn_bw_beta]), prelu_a)
    return y[:, None]


# speedup vs baseline: 4.4451x; 1.0003x over previous
import functools

import jax
import jax.numpy as jnp
from jax import lax
from jax.experimental import pallas as pl
from jax.experimental.pallas import tpu as pltpu
from jax.experimental.pallas import tpu_sc as plsc

_N = 10000
_NC = 2
_NS = 16


def _scmean_body(x2, i2, ones_hbm, zf_hbm, zn_hbm,
                 sums_out, cnts_out,
                 xbuf, ibuf, ones_v, acc, cnt):
    E = x2.shape[1]
    epw = E // _NS
    W = 1000
    nwin = epw // W
    c = lax.axis_index("c")
    s = lax.axis_index("s")

    pltpu.sync_copy(ones_hbm, ones_v)
    @pl.when(s == 0)
    def _init():
        pltpu.sync_copy(zf_hbm, acc)
        pltpu.sync_copy(zn_hbm, cnt)
    plsc.subcore_barrier()

    base = s * epw
    for w in range(nwin):
        off = base + w * W
        pltpu.sync_copy(i2.at[c, pl.ds(off, W)], ibuf)
        pltpu.sync_copy(x2.at[c, pl.ds(off, W), :], xbuf)
        pltpu.sync_copy(xbuf, acc.at[ibuf], add=True)
        pltpu.sync_copy(ones_v, cnt.at[ibuf], add=True)
    plsc.subcore_barrier()

    @pl.when(s == 0)
    def _flush():
        pltpu.sync_copy(acc, sums_out.at[c])
        pltpu.sync_copy(cnt, cnts_out.at[c])


def _scmean(x2, a0, b0):
    _, E, F = x2.shape
    W = 1000
    ones_hbm = jnp.ones((W,), jnp.float32)
    zf_hbm = jnp.zeros((_N, F), jnp.float32)
    zn_hbm = jnp.zeros((_N,), jnp.float32)
    i2 = jnp.stack([a0, b0])
    k = pl.kernel(
        _scmean_body,
        out_type=[jax.ShapeDtypeStruct((2, _N, F), jnp.float32),
                  jax.ShapeDtypeStruct((2, _N), jnp.float32)],
        mesh=plsc.VectorSubcoreMesh(core_axis_name="c", subcore_axis_name="s"),
        compiler_params=pltpu.CompilerParams(use_tc_tiling_on_sc=False),
        scratch_types=[
            pltpu.VMEM((W, F), jnp.float32),
            pltpu.VMEM((W,), jnp.int32),
            pltpu.VMEM((W,), jnp.float32),
            pltpu.VMEM_SHARED((_N, F), jnp.float32),
            pltpu.VMEM_SHARED((_N,), jnp.float32),
        ],
    )
    sums, cnts = k(x2, i2, ones_hbm, zf_hbm, zn_hbm)
    return sums[0], sums[1], cnts[0], cnts[1]


def _scmax_body(u_fbt, tm_fb, ig2, im2, ninf_hbm, zpart_out,
                ubuf, gbuf, ib1, ib0, acc):
    E = ig2.shape[1]
    Q = E // 4
    W = 640
    nwin = Q // W
    ng = W // 16
    c = lax.axis_index("c")
    s = lax.axis_index("s")
    fg = s % 8
    eg = c * 2 + s // 8
    iota = lax.iota(jnp.int32, 16)
    im1 = jnp.maximum(iota - 1, 0)
    fsp = [jnp.full((16,), f, jnp.int32) for f in range(8)]

    for d in range(2):
        pltpu.sync_copy(ninf_hbm, acc)
        qbase = eg * Q

        def _win(w):
            off = qbase + w * W
            pltpu.sync_copy(ig2.at[d, pl.ds(off, W)], ib1)
            pltpu.sync_copy(im2.at[d, pl.ds(off, W)], ib0)
            pltpu.sync_copy(u_fbt.at[d, fg, :, pl.ds(off, W)], ubuf)
            pltpu.sync_copy(tm_fb.at[d, fg].at[ib1], gbuf)

            def _grp(j):
                b16 = j * 16
                a0v = ib0[pl.ds(b16, 16)]
                rows = iota + b16
                zs = []
                for f in range(8):
                    zt = plsc.load_gather(gbuf, [rows, fsp[f]])
                    uv = ubuf[f, pl.ds(b16, 16)]
                    zs.append(zt + uv)

                sk, _ = plsc.sort_key_val(a0v, a0v)
                skm1 = lax.gather(
                    sk, im1[:, None],
                    lax.GatherDimensionNumbers(offset_dims=(),
                                               collapsed_slice_dims=(0,),
                                               start_index_map=(0,)),
                    (1,), mode=lax.GatherScatterMode.PROMISE_IN_BOUNDS)
                dup = jnp.any((sk == skm1) & (iota > 0))

                @pl.when(jnp.logical_not(dup))
                def _fast():
                    for f in range(8):
                        cur = plsc.load_gather(acc, [a0v, fsp[f]])
                        plsc.store_scatter(acc, [a0v, fsp[f]],
                                           jnp.maximum(cur, zs[f]))

                @pl.when(dup)
                def _slow():
                    def _cond(pending):
                        return jnp.any(pending)

                    def _body(pending):
                        fail = jnp.zeros((16,), jnp.bool_)
                        for f in range(8):
                            cur = plsc.load_gather(acc, [a0v, fsp[f]])
                            mx = jnp.maximum(cur, zs[f])
                            plsc.store_scatter(acc, [a0v, fsp[f]], mx,
                                               mask=pending)
                            chk = plsc.load_gather(acc, [a0v, fsp[f]])
                            fail = fail | (chk < mx)
                        return pending & fail

                    lax.while_loop(_cond, _body, jnp.ones((16,), jnp.bool_))

            lax.fori_loop(0, ng, lambda j, _: (_grp(j), 0)[1], 0)

        lax.fori_loop(0, nwin, lambda w, _: (_win(w), 0)[1], 0)
        pltpu.sync_copy(acc, zpart_out.at[d, eg, fg])


def _scmax(u_fbt, tm_fb, ig2, im2):
    E = ig2.shape[1]
    W = 640
    ninf_hbm = jnp.full((_N, 8), -jnp.inf, jnp.float32)
    k = pl.kernel(
        _scmax_body,
        out_type=jax.ShapeDtypeStruct((2, 4, 8, _N, 8), jnp.float32),
        mesh=plsc.VectorSubcoreMesh(core_axis_name="c", subcore_axis_name="s"),
        compiler_params=pltpu.CompilerParams(use_tc_tiling_on_sc=False,
                                             needs_layout_passes=False),
        scratch_types=[
            pltpu.VMEM((8, W), jnp.float32),
            pltpu.VMEM((W, 8), jnp.float32),
            pltpu.VMEM((W,), jnp.int32),
            pltpu.VMEM((W,), jnp.int32),
            pltpu.VMEM((_N, 8), jnp.float32),
        ],
    )
    return k(u_fbt, tm_fb, ig2, im2, ninf_hbm)


def _scg_body(tc2, sc2, igf, imf, g_out, s_out, gb, sb, ibg, ibm):
    E = g_out.shape[1]
    epw = E // _NS
    W = 400
    nwin = epw // W
    c = lax.axis_index("c")
    s = lax.axis_index("s")
    base = s * epw

    def _win(w):
        off = base + w * W
        pltpu.sync_copy(igf.at[pl.ds(c * E + off, W)], ibg)
        pltpu.sync_copy(imf.at[pl.ds(c * E + off, W)], ibm)
        pltpu.sync_copy(tc2.at[c].at[ibg], gb)
        pltpu.sync_copy(sc2.at[c].at[ibm], sb)
        pltpu.sync_copy(gb, g_out.at[c, pl.ds(off, W), :])
        pltpu.sync_copy(sb, s_out.at[c, pl.ds(off, W), :])

    lax.fori_loop(0, nwin, lambda w, _: (_win(w), 0)[1], 0)


def _scgather2(tc2, sc2, ig2, im2):
    E = ig2.shape[1]
    C = tc2.shape[2]
    W = 400
    igf = ig2.reshape(2 * E)
    imf = im2.reshape(2 * E)
    k = pl.kernel(
        _scg_body,
        out_type=[jax.ShapeDtypeStruct((2, E, C), jnp.float32),
                  jax.ShapeDtypeStruct((2, E, C), jnp.float32)],
        mesh=plsc.VectorSubcoreMesh(core_axis_name="c", subcore_axis_name="s"),
        compiler_params=pltpu.CompilerParams(use_tc_tiling_on_sc=True,
                                             needs_layout_passes=False),
        scratch_types=[
            pltpu.VMEM((W, C), jnp.float32),
            pltpu.VMEM((W, C), jnp.float32),
            pltpu.VMEM((W,), jnp.int32),
            pltpu.VMEM((W,), jnp.int32),
        ],
    )
    return k(tc2, sc2, igf, imf)


def _conv_bn_prelu_kernel(x_ref, g_ref, s_ref, w_ref, bc_ref, gam_ref,
                          bet_ref, a_ref, o_ref):
    d = pl.program_id(0)
    g_id = pl.program_id(1)
    xb = x_ref[0, 0]
    y = lax.dot_general(xb, w_ref[...], (((1,), (1,)), ((), ())),
                        preferred_element_type=jnp.float32)
    y = y + bc_ref[...]
    y = y + g_ref[0, 0] + s_ref[0, 0]
    mu = jnp.mean(y)
    var = jnp.mean(y * y) - mu * mu
    g = gam_ref[d, g_id]
    b = bet_ref[d, g_id]
    yn = g * (y - mu) * jax.lax.rsqrt(var + 1e-5) + b
    a = a_ref[0]
    o_ref[0, 0] = jnp.where(yn >= 0, yn, a * yn)


def _conv_bn_prelu(x2, G, S, Wc1, b_conv, gamma, beta, prelu_a):
    D, E, F = x2.shape
    C = G.shape[2]
    bs = E // C
    x4 = x2.reshape(D, C, bs, F)
    G4 = G.reshape(D, C, bs, C)
    S4 = S.reshape(D, C, bs, C)
    out = pl.pallas_call(
        _conv_bn_prelu_kernel,
        grid=(D, C),
        in_specs=[
            pl.BlockSpec((1, 1, bs, F), lambda d, g: (d, g, 0, 0)),
            pl.BlockSpec((1, 1, bs, C), lambda d, g: (d, g, 0, 0)),
            pl.BlockSpec((1, 1, bs, C), lambda d, g: (d, g, 0, 0)),
            pl.BlockSpec((C, F), lambda d, g: (0, 0)),
            pl.BlockSpec((1, C), lambda d, g: (0, 0)),
            pl.BlockSpec(memory_space=pltpu.SMEM),
            pl.BlockSpec(memory_space=pltpu.SMEM),
            pl.BlockSpec(memory_space=pltpu.SMEM),
        ],
        out_specs=pl.BlockSpec((1, 1, bs, C), lambda d, g: (d, g, 0, 0)),
        out_shape=jax.ShapeDtypeStruct((D, C, bs, C), jnp.float32),
    )(x4, G4, S4, Wc1, b_conv.reshape(1, C), gamma, beta, prelu_a.reshape(1))
    return out.reshape(D, 1, E, C)


def kernel(xs, A_to_B_edge_idx, B_to_A_edge_idx, W_max, b_max, W_conv, b_conv,
           bn_fw_gamma, bn_fw_beta, bn_bw_gamma, bn_bw_beta, prelu_a):
    E = xs.shape[2]
    x0 = xs[0, 0]
    x1 = xs[1, 0]
    a0 = A_to_B_edge_idx[0, 0]
    a1 = A_to_B_edge_idx[0, 1]
    b0 = B_to_A_edge_idx[0, 0]
    b1 = B_to_A_edge_idx[0, 1]

    s0, s1, c0, c1 = _scmean(xs.reshape(2, E, -1), a0, b0)
    m0 = s0 / jnp.clip(c0, 1.0)[:, None]
    m1 = s1 / jnp.clip(c1, 1.0)[:, None]

    F = x0.shape[1]
    Wm1, Wm2 = W_max[:, :F], W_max[:, F:]
    t1m = m1 @ Wm2.T + b_max
    t0m = m0 @ Wm2.T + b_max

    u = jnp.einsum('def,gf->dge', xs.reshape(2, E, -1), Wm1)
    u_fbt = u.reshape(2, 8, 8, E)
    tm = jnp.stack([t1m, t0m])
    tm_fb = tm.reshape(2, _N, 8, 8).transpose(0, 2, 1, 3)
    ig2 = jnp.stack([a1, b1])
    im2 = jnp.stack([a0, b0])
    zpart = _scmax(u_fbt, tm_fb, ig2, im2)
    zmax = zpart.max(axis=1).transpose(0, 2, 1, 3).reshape(2, _N, 64)
    cnt2 = jnp.stack([c0, c1])
    zmax = jnp.where(cnt2[:, :, None] > 0, zmax, 0.0)
    zmax_fw, zmax_bw = zmax[0], zmax[1]

    Wc1, Wc2, Wc3 = W_conv[:, :F], W_conv[:, F:2 * F], W_conv[:, 2 * F:]
    t1c = m1 @ Wc2.T
    t0c = m0 @ Wc2.T
    sf = zmax_fw @ Wc3.T
    sb = zmax_bw @ Wc3.T

    tc2 = jnp.stack([t1c, t0c])
    sc2 = jnp.stack([sf, sb])
    G, S = _scgather2(tc2, sc2, ig2, im2)

    return _conv_bn_prelu(xs.reshape(2, E, -1), G, S, Wc1, b_conv,
                          jnp.stack([bn_fw_gamma, bn_bw_gamma]),
                          jnp.stack([bn_fw_beta, bn_bw_beta]), prelu_a)


# speedup vs baseline: 5.4991x; 1.2371x over previous
import functools

import jax
import jax.numpy as jnp
from jax import lax
from jax.experimental import pallas as pl
from jax.experimental.pallas import tpu as pltpu
from jax.experimental.pallas import tpu_sc as plsc

_N = 10000
_NC = 2
_NS = 16


def _scmean_body(x2, i2, ones_hbm, zf_hbm, zn_hbm,
                 sums_out, cnts_out,
                 xbuf, ibuf, ones_v, acc, cnt):
    E = x2.shape[1]
    epw = E // _NS
    W = 1000
    nwin = epw // W
    c = lax.axis_index("c")
    s = lax.axis_index("s")

    pltpu.sync_copy(ones_hbm, ones_v)
    @pl.when(s == 0)
    def _init():
        pltpu.sync_copy(zf_hbm, acc)
        pltpu.sync_copy(zn_hbm, cnt)
    plsc.subcore_barrier()

    base = s * epw
    for w in range(nwin):
        off = base + w * W
        pltpu.sync_copy(i2.at[c, pl.ds(off, W)], ibuf)
        pltpu.sync_copy(x2.at[c, pl.ds(off, W), :], xbuf)
        pltpu.sync_copy(xbuf, acc.at[ibuf], add=True)
        pltpu.sync_copy(ones_v, cnt.at[ibuf], add=True)
    plsc.subcore_barrier()

    @pl.when(s == 0)
    def _flush():
        pltpu.sync_copy(acc, sums_out.at[c])
        pltpu.sync_copy(cnt, cnts_out.at[c])


def _scmean(x2, a0, b0):
    _, E, F = x2.shape
    W = 1000
    ones_hbm = jnp.ones((W,), jnp.float32)
    zf_hbm = jnp.zeros((_N, F), jnp.float32)
    zn_hbm = jnp.zeros((_N,), jnp.float32)
    i2 = jnp.stack([a0, b0])
    k = pl.kernel(
        _scmean_body,
        out_type=[jax.ShapeDtypeStruct((2, _N, F), jnp.float32),
                  jax.ShapeDtypeStruct((2, _N), jnp.float32)],
        mesh=plsc.VectorSubcoreMesh(core_axis_name="c", subcore_axis_name="s"),
        compiler_params=pltpu.CompilerParams(use_tc_tiling_on_sc=False),
        scratch_types=[
            pltpu.VMEM((W, F), jnp.float32),
            pltpu.VMEM((W,), jnp.int32),
            pltpu.VMEM((W,), jnp.float32),
            pltpu.VMEM_SHARED((_N, F), jnp.float32),
            pltpu.VMEM_SHARED((_N,), jnp.float32),
        ],
    )
    sums, cnts = k(x2, i2, ones_hbm, zf_hbm, zn_hbm)
    return sums[0], sums[1], cnts[0], cnts[1]


def _scmax_body(u_fbt, tm_fb, ig2, im2, ninf_hbm, zpart_out,
                ubuf, gbuf, ib1, ib0, acc):
    E = ig2.shape[1]
    Q = E // 4
    W = 640
    nwin = Q // W
    ng = W // 16
    c = lax.axis_index("c")
    s = lax.axis_index("s")
    fg = s % 8
    eg = c * 2 + s // 8
    iota = lax.iota(jnp.int32, 16)
    im1 = jnp.maximum(iota - 1, 0)
    fsp = [jnp.full((16,), f, jnp.int32) for f in range(8)]

    for d in range(2):
        pltpu.sync_copy(ninf_hbm, acc)
        qbase = eg * Q

        def _win(w):
            off = qbase + w * W
            pltpu.sync_copy(ig2.at[d, pl.ds(off, W)], ib1)
            pltpu.sync_copy(im2.at[d, pl.ds(off, W)], ib0)
            pltpu.sync_copy(u_fbt.at[d, fg, :, pl.ds(off, W)], ubuf)
            pltpu.sync_copy(tm_fb.at[d, fg].at[ib1], gbuf)

            def _grp(j):
                b16 = j * 16
                a0v = ib0[pl.ds(b16, 16)]
                rows = iota + b16
                zs = []
                for f in range(8):
                    zt = plsc.load_gather(gbuf, [rows, fsp[f]])
                    uv = ubuf[f, pl.ds(b16, 16)]
                    zs.append(zt + uv)

                sk, _ = plsc.sort_key_val(a0v, a0v)
                skm1 = lax.gather(
                    sk, im1[:, None],
                    lax.GatherDimensionNumbers(offset_dims=(),
                                               collapsed_slice_dims=(0,),
                                               start_index_map=(0,)),
                    (1,), mode=lax.GatherScatterMode.PROMISE_IN_BOUNDS)
                dup = jnp.any((sk == skm1) & (iota > 0))

                @pl.when(jnp.logical_not(dup))
                def _fast():
                    for f in range(8):
                        cur = plsc.load_gather(acc, [a0v, fsp[f]])
                        plsc.store_scatter(acc, [a0v, fsp[f]],
                                           jnp.maximum(cur, zs[f]))

                @pl.when(dup)
                def _slow():
                    def _cond(pending):
                        return jnp.any(pending)

                    def _body(pending):
                        fail = jnp.zeros((16,), jnp.bool_)
                        for f in range(8):
                            cur = plsc.load_gather(acc, [a0v, fsp[f]])
                            mx = jnp.maximum(cur, zs[f])
                            plsc.store_scatter(acc, [a0v, fsp[f]], mx,
                                               mask=pending)
                            chk = plsc.load_gather(acc, [a0v, fsp[f]])
                            fail = fail | (chk < mx)
                        return pending & fail

                    lax.while_loop(_cond, _body, jnp.ones((16,), jnp.bool_))

            lax.fori_loop(0, ng, lambda j, _: (_grp(j), 0)[1], 0)

        lax.fori_loop(0, nwin, lambda w, _: (_win(w), 0)[1], 0)
        pltpu.sync_copy(acc, zpart_out.at[d, eg, fg])


def _scmax(u_fbt, tm_fb, ig2, im2):
    E = ig2.shape[1]
    W = 640
    ninf_hbm = jnp.full((_N, 8), -jnp.inf, jnp.float32)
    k = pl.kernel(
        _scmax_body,
        out_type=jax.ShapeDtypeStruct((2, 4, 8, _N, 8), jnp.float32),
        mesh=plsc.VectorSubcoreMesh(core_axis_name="c", subcore_axis_name="s"),
        compiler_params=pltpu.CompilerParams(use_tc_tiling_on_sc=False,
                                             needs_layout_passes=False),
        scratch_types=[
            pltpu.VMEM((8, W), jnp.float32),
            pltpu.VMEM((W, 8), jnp.float32),
            pltpu.VMEM((W,), jnp.int32),
            pltpu.VMEM((W,), jnp.int32),
            pltpu.VMEM((_N, 8), jnp.float32),
        ],
    )
    return k(u_fbt, tm_fb, ig2, im2, ninf_hbm)


def _scg_body(tc2, sc2, igf, imf, g_out, s_out, gb, sb, ibg, ibm):
    E = g_out.shape[1]
    epw = E // _NS
    W = 400
    nwin = epw // W
    c = lax.axis_index("c")
    s = lax.axis_index("s")
    base = s * epw

    def _win(w):
        off = base + w * W
        pltpu.sync_copy(igf.at[pl.ds(c * E + off, W)], ibg)
        pltpu.sync_copy(imf.at[pl.ds(c * E + off, W)], ibm)
        pltpu.sync_copy(tc2.at[c].at[ibg], gb)
        pltpu.sync_copy(sc2.at[c].at[ibm], sb)
        pltpu.sync_copy(gb, g_out.at[c, pl.ds(off, W), :])
        pltpu.sync_copy(sb, s_out.at[c, pl.ds(off, W), :])

    lax.fori_loop(0, nwin, lambda w, _: (_win(w), 0)[1], 0)


def _scgather2(tc2, sc2, ig2, im2):
    E = ig2.shape[1]
    C = tc2.shape[2]
    W = 400
    igf = ig2.reshape(2 * E)
    imf = im2.reshape(2 * E)
    k = pl.kernel(
        _scg_body,
        out_type=[jax.ShapeDtypeStruct((2, E, C), jnp.float32),
                  jax.ShapeDtypeStruct((2, E, C), jnp.float32)],
        mesh=plsc.VectorSubcoreMesh(core_axis_name="c", subcore_axis_name="s"),
        compiler_params=pltpu.CompilerParams(use_tc_tiling_on_sc=True,
                                             needs_layout_passes=False),
        scratch_types=[
            pltpu.VMEM((W, C), jnp.float32),
            pltpu.VMEM((W, C), jnp.float32),
            pltpu.VMEM((W,), jnp.int32),
            pltpu.VMEM((W,), jnp.int32),
        ],
    )
    return k(tc2, sc2, igf, imf)


def _conv_bn_prelu_kernel(x_ref, g_ref, s_ref, w_ref, bc_ref, gam_ref,
                          bet_ref, a_ref, o_ref):
    d = pl.program_id(0)
    blk = pl.program_id(1)
    xb = x_ref[0]
    y = lax.dot_general(xb, w_ref[...], (((1,), (1,)), ((), ())),
                        preferred_element_type=jnp.float32)
    y = y + bc_ref[...]
    y = y + g_ref[0] + s_ref[0]
    a = a_ref[0]
    for h in range(2):
        yh = y[h * 2500:(h + 1) * 2500, :]
        mu = jnp.mean(yh)
        var = jnp.mean(yh * yh) - mu * mu
        g = gam_ref[d, 2 * blk + h]
        b = bet_ref[d, 2 * blk + h]
        yn = g * (yh - mu) * jax.lax.rsqrt(var + 1e-5) + b
        o_ref[0, h * 2500:(h + 1) * 2500, :] = jnp.where(yn >= 0, yn, a * yn)


def _conv_bn_prelu(x2, G, S, Wc1, b_conv, gamma, beta, prelu_a):
    D, E, F = x2.shape
    C = G.shape[2]
    BS = 5000
    nb = E // BS
    out = pl.pallas_call(
        _conv_bn_prelu_kernel,
        grid=(D, nb),
        in_specs=[
            pl.BlockSpec((1, BS, F), lambda d, g: (d, g, 0)),
            pl.BlockSpec((1, BS, C), lambda d, g: (d, g, 0)),
            pl.BlockSpec((1, BS, C), lambda d, g: (d, g, 0)),
            pl.BlockSpec((C, F), lambda d, g: (0, 0)),
            pl.BlockSpec((1, C), lambda d, g: (0, 0)),
            pl.BlockSpec(memory_space=pltpu.SMEM),
            pl.BlockSpec(memory_space=pltpu.SMEM),
            pl.BlockSpec(memory_space=pltpu.SMEM),
        ],
        out_specs=pl.BlockSpec((1, BS, C), lambda d, g: (d, g, 0)),
        out_shape=jax.ShapeDtypeStruct((D, E, C), jnp.float32),
    )(x2, G, S, Wc1, b_conv.reshape(1, C), gamma, beta, prelu_a.reshape(1))
    return out.reshape(D, 1, E, C)


def kernel(xs, A_to_B_edge_idx, B_to_A_edge_idx, W_max, b_max, W_conv, b_conv,
           bn_fw_gamma, bn_fw_beta, bn_bw_gamma, bn_bw_beta, prelu_a):
    E = xs.shape[2]
    x0 = xs[0, 0]
    x1 = xs[1, 0]
    a0 = A_to_B_edge_idx[0, 0]
    a1 = A_to_B_edge_idx[0, 1]
    b0 = B_to_A_edge_idx[0, 0]
    b1 = B_to_A_edge_idx[0, 1]

    s0, s1, c0, c1 = _scmean(xs.reshape(2, E, -1), a0, b0)
    m0 = s0 / jnp.clip(c0, 1.0)[:, None]
    m1 = s1 / jnp.clip(c1, 1.0)[:, None]

    F = x0.shape[1]
    Wm1, Wm2 = W_max[:, :F], W_max[:, F:]
    t1m = m1 @ Wm2.T + b_max
    t0m = m0 @ Wm2.T + b_max

    u = jnp.einsum('def,gf->dge', xs.reshape(2, E, -1), Wm1)
    u_fbt = u.reshape(2, 8, 8, E)
    tm = jnp.stack([t1m, t0m])
    tm_fb = tm.reshape(2, _N, 8, 8).transpose(0, 2, 1, 3)
    ig2 = jnp.stack([a1, b1])
    im2 = jnp.stack([a0, b0])
    zpart = _scmax(u_fbt, tm_fb, ig2, im2)
    zmax = zpart.max(axis=1).transpose(0, 2, 1, 3).reshape(2, _N, 64)
    cnt2 = jnp.stack([c0, c1])
    zmax = jnp.where(cnt2[:, :, None] > 0, zmax, 0.0)
    zmax_fw, zmax_bw = zmax[0], zmax[1]

    Wc1, Wc2, Wc3 = W_conv[:, :F], W_conv[:, F:2 * F], W_conv[:, 2 * F:]
    t1c = m1 @ Wc2.T
    t0c = m0 @ Wc2.T
    sf = zmax_fw @ Wc3.T
    sb = zmax_bw @ Wc3.T

    tc2 = jnp.stack([t1c, t0c])
    sc2 = jnp.stack([sf, sb])
    G, S = _scgather2(tc2, sc2, ig2, im2)

    return _conv_bn_prelu(xs.reshape(2, E, -1), G, S, Wc1, b_conv,
                          jnp.stack([bn_fw_gamma, bn_bw_gamma]),
                          jnp.stack([bn_fw_beta, bn_bw_beta]), prelu_a)


# speedup vs baseline: 5.7606x; 1.0475x over previous
import functools

import jax
import jax.numpy as jnp
from jax import lax
from jax.experimental import pallas as pl
from jax.experimental.pallas import tpu as pltpu
from jax.experimental.pallas import tpu_sc as plsc

_N = 10000
_NC = 2
_NS = 16


def _scmean_body(x2, i2, ones_hbm, zf_hbm, zn_hbm,
                 sums_out, cnts_out,
                 xbuf, ibuf, ones_v, acc, cnt):
    E = x2.shape[1]
    epw = E // _NS
    W = 1000
    nwin = epw // W
    c = lax.axis_index("c")
    s = lax.axis_index("s")

    pltpu.sync_copy(ones_hbm, ones_v)
    @pl.when(s == 0)
    def _init():
        pltpu.sync_copy(zf_hbm, acc)
        pltpu.sync_copy(zn_hbm, cnt)
    plsc.subcore_barrier()

    base = s * epw
    for w in range(nwin):
        off = base + w * W
        pltpu.sync_copy(i2.at[c, pl.ds(off, W)], ibuf)
        pltpu.sync_copy(x2.at[c, pl.ds(off, W), :], xbuf)
        pltpu.sync_copy(xbuf, acc.at[ibuf], add=True)
        pltpu.sync_copy(ones_v, cnt.at[ibuf], add=True)
    plsc.subcore_barrier()

    @pl.when(s == 0)
    def _flush():
        pltpu.sync_copy(acc, sums_out.at[c])
        pltpu.sync_copy(cnt, cnts_out.at[c])


def _scmean(x2, a0, b0):
    _, E, F = x2.shape
    W = 1000
    ones_hbm = jnp.ones((W,), jnp.float32)
    zf_hbm = jnp.zeros((_N, F), jnp.float32)
    zn_hbm = jnp.zeros((_N,), jnp.float32)
    i2 = jnp.stack([a0, b0])
    k = pl.kernel(
        _scmean_body,
        out_type=[jax.ShapeDtypeStruct((2, _N, F), jnp.float32),
                  jax.ShapeDtypeStruct((2, _N), jnp.float32)],
        mesh=plsc.VectorSubcoreMesh(core_axis_name="c", subcore_axis_name="s"),
        compiler_params=pltpu.CompilerParams(use_tc_tiling_on_sc=False),
        scratch_types=[
            pltpu.VMEM((W, F), jnp.float32),
            pltpu.VMEM((W,), jnp.int32),
            pltpu.VMEM((W,), jnp.float32),
            pltpu.VMEM_SHARED((_N, F), jnp.float32),
            pltpu.VMEM_SHARED((_N,), jnp.float32),
        ],
    )
    sums, cnts = k(x2, i2, ones_hbm, zf_hbm, zn_hbm)
    return sums[0], sums[1], cnts[0], cnts[1]


def _scmax_body(u_fbt, tm_fb, ig2, im2, ninf_hbm, zpart_out,
                ubuf, gbuf, ib1, ib0, acc):
    E = ig2.shape[1]
    Q = E // 4
    W = 640
    nwin = Q // W
    ng = W // 16
    c = lax.axis_index("c")
    s = lax.axis_index("s")
    fg = s % 8
    eg = c * 2 + s // 8
    iota = lax.iota(jnp.int32, 16)
    im1 = jnp.maximum(iota - 1, 0)
    fsp = [jnp.full((16,), f, jnp.int32) for f in range(8)]

    for d in range(2):
        pltpu.sync_copy(ninf_hbm, acc)
        qbase = eg * Q

        def _win(w):
            off = qbase + w * W
            pltpu.sync_copy(ig2.at[d, pl.ds(off, W)], ib1)
            pltpu.sync_copy(im2.at[d, pl.ds(off, W)], ib0)
            pltpu.sync_copy(u_fbt.at[d, fg, :, pl.ds(off // 128, W // 128), :],
                            ubuf)
            pltpu.sync_copy(tm_fb.at[d, fg].at[ib1], gbuf)

            def _grp(j):
                b16 = j * 16
                a0v = ib0[pl.ds(b16, 16)]
                arow = lax.shift_right_logical(a0v, 4)
                acol = (a0v & 15) * 8
                rows = iota + b16
                zs = []
                for f in range(8):
                    zt = plsc.load_gather(gbuf, [rows, fsp[f]])
                    uv = ubuf[f, j >> 3, pl.ds((b16 % 128), 16)]
                    zs.append(zt + uv)

                sk, _ = plsc.sort_key_val(a0v, a0v)
                skm1 = lax.gather(
                    sk, im1[:, None],
                    lax.GatherDimensionNumbers(offset_dims=(),
                                               collapsed_slice_dims=(0,),
                                               start_index_map=(0,)),
                    (1,), mode=lax.GatherScatterMode.PROMISE_IN_BOUNDS)
                dup = jnp.any((sk == skm1) & (iota > 0))

                @pl.when(jnp.logical_not(dup))
                def _fast():
                    for f in range(8):
                        cur = plsc.load_gather(acc, [arow, acol + f])
                        plsc.store_scatter(acc, [arow, acol + f],
                                           jnp.maximum(cur, zs[f]))

                @pl.when(dup)
                def _slow():
                    def _cond(pending):
                        return jnp.any(pending)

                    def _body(pending):
                        fail = jnp.zeros((16,), jnp.bool_)
                        for f in range(8):
                            cur = plsc.load_gather(acc, [arow, acol + f])
                            mx = jnp.maximum(cur, zs[f])
                            plsc.store_scatter(acc, [arow, acol + f], mx,
                                               mask=pending)
                            chk = plsc.load_gather(acc, [arow, acol + f])
                            fail = fail | (chk < mx)
                        return pending & fail

                    lax.while_loop(_cond, _body, jnp.ones((16,), jnp.bool_))

            lax.fori_loop(0, ng, lambda j, _: (_grp(j), 0)[1], 0)

        lax.fori_loop(0, nwin, lambda w, _: (_win(w), 0)[1], 0)
        pltpu.sync_copy(acc, zpart_out.at[d, eg, fg])


def _scmax(u_fbt, tm_fb, ig2, im2):
    E = ig2.shape[1]
    W = 640
    ninf_hbm = jnp.full((_N * 8 // 128, 128), -jnp.inf, jnp.float32)
    k = pl.kernel(
        _scmax_body,
        out_type=jax.ShapeDtypeStruct((2, 4, 8, _N * 8 // 128, 128),
                                      jnp.float32),
        mesh=plsc.VectorSubcoreMesh(core_axis_name="c", subcore_axis_name="s"),
        compiler_params=pltpu.CompilerParams(use_tc_tiling_on_sc=False,
                                             needs_layout_passes=False),
        scratch_types=[
            pltpu.VMEM((8, W // 128, 128), jnp.float32),
            pltpu.VMEM((W, 8), jnp.float32),
            pltpu.VMEM((W,), jnp.int32),
            pltpu.VMEM((W,), jnp.int32),
            pltpu.VMEM((_N * 8 // 128, 128), jnp.float32),
        ],
    )
    return k(u_fbt, tm_fb, ig2, im2, ninf_hbm)


def _scg_body(tc2, sc2, igf, imf, g_out, s_out, gb, sb, ibg, ibm):
    E = g_out.shape[1]
    epw = E // _NS
    W = 400
    nwin = epw // W
    c = lax.axis_index("c")
    s = lax.axis_index("s")
    base = s * epw

    def _win(w):
        off = base + w * W
        pltpu.sync_copy(igf.at[pl.ds(c * E + off, W)], ibg)
        pltpu.sync_copy(imf.at[pl.ds(c * E + off, W)], ibm)
        pltpu.sync_copy(tc2.at[c].at[ibg], gb)
        pltpu.sync_copy(sc2.at[c].at[ibm], sb)
        pltpu.sync_copy(gb, g_out.at[c, pl.ds(off, W), :])
        pltpu.sync_copy(sb, s_out.at[c, pl.ds(off, W), :])

    lax.fori_loop(0, nwin, lambda w, _: (_win(w), 0)[1], 0)


def _scgather2(tc2, sc2, ig2, im2):
    E = ig2.shape[1]
    C = tc2.shape[2]
    W = 400
    igf = ig2.reshape(2 * E)
    imf = im2.reshape(2 * E)
    k = pl.kernel(
        _scg_body,
        out_type=[jax.ShapeDtypeStruct((2, E, C), jnp.float32),
                  jax.ShapeDtypeStruct((2, E, C), jnp.float32)],
        mesh=plsc.VectorSubcoreMesh(core_axis_name="c", subcore_axis_name="s"),
        compiler_params=pltpu.CompilerParams(use_tc_tiling_on_sc=True,
                                             needs_layout_passes=False),
        scratch_types=[
            pltpu.VMEM((W, C), jnp.float32),
            pltpu.VMEM((W, C), jnp.float32),
            pltpu.VMEM((W,), jnp.int32),
            pltpu.VMEM((W,), jnp.int32),
        ],
    )
    return k(tc2, sc2, igf, imf)


def _conv_bn_prelu_kernel(x_ref, g_ref, s_ref, w_ref, bc_ref, gam_ref,
                          bet_ref, a_ref, o_ref):
    d = pl.program_id(0)
    blk = pl.program_id(1)
    xb = x_ref[0]
    y = lax.dot_general(xb, w_ref[...], (((1,), (1,)), ((), ())),
                        preferred_element_type=jnp.float32)
    y = y + bc_ref[...]
    y = y + g_ref[0] + s_ref[0]
    a = a_ref[0]
    for h in range(2):
        yh = y[h * 2500:(h + 1) * 2500, :]
        mu = jnp.mean(yh)
        var = jnp.mean(yh * yh) - mu * mu
        g = gam_ref[d, 2 * blk + h]
        b = bet_ref[d, 2 * blk + h]
        yn = g * (yh - mu) * jax.lax.rsqrt(var + 1e-5) + b
        o_ref[0, h * 2500:(h + 1) * 2500, :] = jnp.where(yn >= 0, yn, a * yn)


def _conv_bn_prelu(x2, G, S, Wc1, b_conv, gamma, beta, prelu_a):
    D, E, F = x2.shape
    C = G.shape[2]
    BS = 5000
    nb = E // BS
    out = pl.pallas_call(
        _conv_bn_prelu_kernel,
        grid=(D, nb),
        in_specs=[
            pl.BlockSpec((1, BS, F), lambda d, g: (d, g, 0)),
            pl.BlockSpec((1, BS, C), lambda d, g: (d, g, 0)),
            pl.BlockSpec((1, BS, C), lambda d, g: (d, g, 0)),
            pl.BlockSpec((C, F), lambda d, g: (0, 0)),
            pl.BlockSpec((1, C), lambda d, g: (0, 0)),
            pl.BlockSpec(memory_space=pltpu.SMEM),
            pl.BlockSpec(memory_space=pltpu.SMEM),
            pl.BlockSpec(memory_space=pltpu.SMEM),
        ],
        out_specs=pl.BlockSpec((1, BS, C), lambda d, g: (d, g, 0)),
        out_shape=jax.ShapeDtypeStruct((D, E, C), jnp.float32),
    )(x2, G, S, Wc1, b_conv.reshape(1, C), gamma, beta, prelu_a.reshape(1))
    return out.reshape(D, 1, E, C)


def kernel(xs, A_to_B_edge_idx, B_to_A_edge_idx, W_max, b_max, W_conv, b_conv,
           bn_fw_gamma, bn_fw_beta, bn_bw_gamma, bn_bw_beta, prelu_a):
    E = xs.shape[2]
    x0 = xs[0, 0]
    x1 = xs[1, 0]
    a0 = A_to_B_edge_idx[0, 0]
    a1 = A_to_B_edge_idx[0, 1]
    b0 = B_to_A_edge_idx[0, 0]
    b1 = B_to_A_edge_idx[0, 1]

    s0, s1, c0, c1 = _scmean(xs.reshape(2, E, -1), a0, b0)
    m0 = s0 / jnp.clip(c0, 1.0)[:, None]
    m1 = s1 / jnp.clip(c1, 1.0)[:, None]

    F = x0.shape[1]
    Wm1, Wm2 = W_max[:, :F], W_max[:, F:]
    t1m = m1 @ Wm2.T + b_max
    t0m = m0 @ Wm2.T + b_max

    u = jnp.einsum('def,gf->dge', xs.reshape(2, E, -1), Wm1)
    u_fbt = u.reshape(2, 8, 8, E // 128, 128)
    tm = jnp.stack([t1m, t0m])
    tm_fb = tm.reshape(2, _N, 8, 8).transpose(0, 2, 1, 3)
    ig2 = jnp.stack([a1, b1])
    im2 = jnp.stack([a0, b0])
    zpart = _scmax(u_fbt, tm_fb, ig2, im2)
    zmax = (zpart.max(axis=1).reshape(2, 8, _N // 16, 16, 8)
            .transpose(0, 2, 3, 1, 4).reshape(2, _N, 64))
    cnt2 = jnp.stack([c0, c1])
    zmax = jnp.where(cnt2[:, :, None] > 0, zmax, 0.0)
    zmax_fw, zmax_bw = zmax[0], zmax[1]

    Wc1, Wc2, Wc3 = W_conv[:, :F], W_conv[:, F:2 * F], W_conv[:, 2 * F:]
    t1c = m1 @ Wc2.T
    t0c = m0 @ Wc2.T
    sf = zmax_fw @ Wc3.T
    sb = zmax_bw @ Wc3.T

    tc2 = jnp.stack([t1c, t0c])
    sc2 = jnp.stack([sf, sb])
    G, S = _scgather2(tc2, sc2, ig2, im2)

    return _conv_bn_prelu(xs.reshape(2, E, -1), G, S, Wc1, b_conv,
                          jnp.stack([bn_fw_gamma, bn_bw_gamma]),
                          jnp.stack([bn_fw_beta, bn_bw_beta]), prelu_a)
